# baseline (speedup 1.0000x reference)
import functools

import jax
import jax.numpy as jnp
from jax import lax
from jax.experimental import pallas as pl
from jax.experimental.pallas import tpu as pltpu
from jax.experimental.pallas import tpu_sc as plsc

SCALE = 511.0
B, C, H, W = 2, 128, 512, 512
HW = H * W
N = 131072

NC, NS, L = 2, 16, 16
NW = NC * NS
PTS_PER_W = (B * N) // NW
P = 128
CHUNKS = PTS_PER_W // P

HCHUNK = 4096
NCHUNK = 2048


def _tr_in_body(g_ref, t_ref):
    t_ref[...] = g_ref[0].T


def _build_table(grid3):
    nblk = HW // HCHUNK
    return pl.pallas_call(
        _tr_in_body,
        grid=(B, nblk),
        in_specs=[pl.BlockSpec((1, C, HCHUNK), lambda b, j: (b, 0, j))],
        out_specs=pl.BlockSpec((HCHUNK, C), lambda b, j: (b * nblk + j, 0)),
        out_shape=jax.ShapeDtypeStruct((B * HW, C), jnp.float32),
    )(grid3)


def _tr_out_body(p_ref, o_ref):
    o_ref[0] = p_ref[0].T


def _transpose_out(pm):
    nblk = N // NCHUNK
    return pl.pallas_call(
        _tr_out_body,
        grid=(B, nblk),
        in_specs=[pl.BlockSpec((1, NCHUNK, C), lambda b, j: (b, j, 0))],
        out_specs=pl.BlockSpec((1, C, NCHUNK), lambda b, j: (b, 0, j)),
        out_shape=jax.ShapeDtypeStruct((B, C, N), jnp.float32),
    )(pm)


@functools.partial(
    pl.kernel,
    out_type=jax.ShapeDtypeStruct((B, C, N), jnp.float32),
    mesh=plsc.VectorSubcoreMesh(core_axis_name="c", subcore_axis_name="s"),
    compiler_params=pltpu.CompilerParams(needs_layout_passes=False),
    scratch_types=[
        pltpu.VMEM((2 * P,), jnp.float32),
        pltpu.VMEM((4 * P,), jnp.int32),
        pltpu.VMEM((P,), jnp.float32),
        pltpu.VMEM((P,), jnp.float32),
        pltpu.VMEM((4 * P, C), jnp.float32),
        pltpu.VMEM((C, P), jnp.float32),
        pltpu.SemaphoreType.DMA,
    ],
)
def _sc_gather(table, pc_hbm, out, cv, iall, wh, ww, rall, oc, sem):
    cid = lax.axis_index("c")
    sid = lax.axis_index("s")
    wid = sid * NC + cid
    b = wid // NS
    lane = wid % NS
    base = lane * PTS_PER_W
    iota = lax.iota(jnp.int32, L)
    boff = b * HW

    def chunk(g, carry):
        n0 = base + g * P
        pltpu.sync_copy(pc_hbm.at[b, pl.ds(2 * n0, 2 * P)], cv)
        for t in range(P // L):
            sl = pl.ds(t * L, L)
            hv = plsc.load_gather(cv, [t * (2 * L) + iota * 2]) * SCALE
            wv = plsc.load_gather(cv, [t * (2 * L) + iota * 2 + 1]) * SCALE
            h0i = hv.astype(jnp.int32)
            w0i = wv.astype(jnp.int32)
            wh[sl] = hv - h0i.astype(jnp.float32)
            ww[sl] = wv - w0i.astype(jnp.float32)
            r0 = boff + h0i * W + w0i
            iall[pl.ds(t * L, L)] = r0
            iall[pl.ds(P + t * L, L)] = r0 + 1
            iall[pl.ds(2 * P + t * L, L)] = r0 + W
            iall[pl.ds(3 * P + t * L, L)] = r0 + (W + 1)
        pltpu.async_copy(table.at[iall], rall, sem).wait()

        def pt(i, carry2):
            iv = jnp.full((L,), i, jnp.int32)
            ah = plsc.load_gather(wh, [iv])
            aw = plsc.load_gather(ww, [iv])
            for t in range(C // L):
                sl = pl.ds(t * L, L)
                f00 = rall[i, sl]
                f01 = rall[P + i, sl]
                f10 = rall[2 * P + i, sl]
                f11 = rall[3 * P + i, sl]
                l0 = f00 + aw * (f01 - f00)
                l1 = f10 + aw * (f11 - f10)
                plsc.store_scatter(oc, [t * L + iota, iv],
                                   l0 + ah * (l1 - l0))
            return carry2

        lax.fori_loop(0, P, pt, 0, unroll=2)
        pltpu.sync_copy(oc, out.at[b, :, pl.ds(n0, P)])
        return carry

    lax.fori_loop(0, CHUNKS, chunk, 0)


def kernel(grid_in, pcds_ind):
    grid3 = grid_in.reshape(B, C, HW)
    table = _build_table(grid3)
    pc = pcds_ind.reshape(B, 2 * N)
    out = _sc_gather(table, pc)
    return out[..., None]


# speedup vs baseline: 1.6582x; 1.6582x over previous
import functools

import jax
import jax.numpy as jnp
from jax import lax
from jax.experimental import pallas as pl
from jax.experimental.pallas import tpu as pltpu
from jax.experimental.pallas import tpu_sc as plsc

SCALE = 511.0
B, C, H, W = 2, 128, 512, 512
HW = H * W
N = 131072

NC, NS, L = 2, 16, 16
NW = NC * NS
PTS_PER_W = (B * N) // NW
P = 128
CHUNKS = PTS_PER_W // P

HCHUNK = 4096
NCHUNK = 2048


def _tr_in_body(g_ref, t_ref):
    t_ref[...] = g_ref[0].T


def _build_table(grid3):
    nblk = HW // HCHUNK
    return pl.pallas_call(
        _tr_in_body,
        grid=(B, nblk),
        in_specs=[pl.BlockSpec((1, C, HCHUNK), lambda b, j: (b, 0, j))],
        out_specs=pl.BlockSpec((HCHUNK, C), lambda b, j: (b * nblk + j, 0)),
        out_shape=jax.ShapeDtypeStruct((B * HW, C), jnp.float32),
    )(grid3)


def _tr_out_body(p_ref, o_ref):
    o_ref[0] = p_ref[0].T


def _transpose_out(pm):
    nblk = N // NCHUNK
    return pl.pallas_call(
        _tr_out_body,
        grid=(B, nblk),
        in_specs=[pl.BlockSpec((1, NCHUNK, C), lambda b, j: (b, j, 0))],
        out_specs=pl.BlockSpec((1, C, NCHUNK), lambda b, j: (b, 0, j)),
        out_shape=jax.ShapeDtypeStruct((B, C, N), jnp.float32),
    )(pm)


@functools.partial(
    pl.kernel,
    out_type=jax.ShapeDtypeStruct((B, N, C), jnp.float32),
    mesh=plsc.VectorSubcoreMesh(core_axis_name="c", subcore_axis_name="s"),
    compiler_params=pltpu.CompilerParams(needs_layout_passes=False),
    scratch_types=[
        pltpu.VMEM((2 * P,), jnp.float32),
        pltpu.VMEM((4 * P,), jnp.int32),
        pltpu.VMEM((P,), jnp.float32),
        pltpu.VMEM((P,), jnp.float32),
        pltpu.VMEM((4 * P, C), jnp.float32),
        pltpu.VMEM((P, C), jnp.float32),
        pltpu.SemaphoreType.DMA,
    ],
)
def _sc_gather(table, pc_hbm, out, cv, iall, wh, ww, rall, opm, sem):
    cid = lax.axis_index("c")
    sid = lax.axis_index("s")
    wid = sid * NC + cid
    b = wid // NS
    lane = wid % NS
    base = lane * PTS_PER_W
    iota = lax.iota(jnp.int32, L)
    boff = b * HW

    def chunk(g, carry):
        n0 = base + g * P
        pltpu.sync_copy(pc_hbm.at[b, pl.ds(2 * n0, 2 * P)], cv)
        for t in range(P // L):
            sl = pl.ds(t * L, L)
            hv = plsc.load_gather(cv, [t * (2 * L) + iota * 2]) * SCALE
            wv = plsc.load_gather(cv, [t * (2 * L) + iota * 2 + 1]) * SCALE
            h0i = hv.astype(jnp.int32)
            w0i = wv.astype(jnp.int32)
            wh[sl] = hv - h0i.astype(jnp.float32)
            ww[sl] = wv - w0i.astype(jnp.float32)
            r0 = boff + h0i * W + w0i
            iall[pl.ds(t * L, L)] = r0
            iall[pl.ds(P + t * L, L)] = r0 + 1
            iall[pl.ds(2 * P + t * L, L)] = r0 + W
            iall[pl.ds(3 * P + t * L, L)] = r0 + (W + 1)
        pltpu.async_copy(table.at[iall], rall, sem).wait()

        def pt(i, carry2):
            iv = jnp.full((L,), i, jnp.int32)
            ah = plsc.load_gather(wh, [iv])
            aw = plsc.load_gather(ww, [iv])
            for t in range(C // L):
                sl = pl.ds(t * L, L)
                f00 = rall[i, sl]
                f01 = rall[P + i, sl]
                f10 = rall[2 * P + i, sl]
                f11 = rall[3 * P + i, sl]
                l0 = f00 + aw * (f01 - f00)
                l1 = f10 + aw * (f11 - f10)
                opm[i, sl] = l0 + ah * (l1 - l0)
            return carry2

        lax.fori_loop(0, P, pt, 0, unroll=2)
        pltpu.sync_copy(opm, out.at[b, pl.ds(n0, P), :])
        return carry

    lax.fori_loop(0, CHUNKS, chunk, 0)


def kernel(grid_in, pcds_ind):
    grid3 = grid_in.reshape(B, C, HW)
    table = _build_table(grid3)
    pc = pcds_ind.reshape(B, 2 * N)
    pm = _sc_gather(table, pc)
    out = _transpose_out(pm)
    return out[..., None]
